# baseline (speedup 1.0000x reference)
import functools

import jax
import jax.numpy as jnp
from jax import lax
from jax.experimental import pallas as pl
from jax.experimental.pallas import tpu as pltpu
from jax.experimental.pallas import tpu_sc as plsc

ALPHA = 0.1
KHOPS = 10
LANES = 16
NTILES = 16
SUB = 128
CP = 48


def _mlp_body(x_ref, w1_ref, b1_ref, w2_ref, b2_ref, h_ref):
    h = jnp.dot(x_ref[...], w1_ref[...], preferred_element_type=jnp.float32)
    h = jnp.maximum(h + b1_ref[...], 0.0)
    h2 = jnp.dot(h, w2_ref[...], preferred_element_type=jnp.float32)
    h2 = h2 + b2_ref[...]
    pad = jnp.zeros((h2.shape[0], CP - h2.shape[1]), jnp.float32)
    h_ref[...] = jnp.concatenate([h2, pad], axis=1)


def _mlp(x_pad, w1t, b1r, w2t, b2r, n_pad, blk):
    f_in = x_pad.shape[1]
    hid = w1t.shape[1]
    c = w2t.shape[1]
    return pl.pallas_call(
        _mlp_body,
        grid=(n_pad // blk,),
        in_specs=[
            pl.BlockSpec((blk, f_in), lambda i: (i, 0)),
            pl.BlockSpec((f_in, hid), lambda i: (0, 0)),
            pl.BlockSpec((1, hid), lambda i: (0, 0)),
            pl.BlockSpec((hid, c), lambda i: (0, 0)),
            pl.BlockSpec((1, c), lambda i: (0, 0)),
        ],
        out_specs=pl.BlockSpec((blk, CP), lambda i: (i, 0)),
        out_shape=jax.ShapeDtypeStruct((n_pad, CP), jnp.float32),
    )(x_pad, w1t, b1r, w2t, b2r)


def _prep_body(h_ref, dg_ref, u0_ref, d2_ref, a0_ref):
    hp = h_ref[...]
    deg = dg_ref[...][:, 0:1] + 1.0
    dinv = lax.rsqrt(deg)
    u0_ref[...] = dinv * hp
    d2_ref[...] = jnp.broadcast_to((1.0 - ALPHA) * dinv * dinv,
                                   (hp.shape[0], 16))
    a0_ref[...] = (ALPHA / (1.0 - ALPHA)) * hp * jnp.sqrt(deg)


def _prep(h_pad, deg16, n_pad, blk):
    return pl.pallas_call(
        _prep_body,
        grid=(n_pad // blk,),
        in_specs=[
            pl.BlockSpec((blk, CP), lambda i: (i, 0)),
            pl.BlockSpec((blk, 16), lambda i: (i, 0)),
        ],
        out_specs=[
            pl.BlockSpec((blk, CP), lambda i: (i, 0)),
            pl.BlockSpec((blk, 16), lambda i: (i, 0)),
            pl.BlockSpec((blk, CP), lambda i: (i, 0)),
        ],
        out_shape=[
            jax.ShapeDtypeStruct((n_pad, CP), jnp.float32),
            jax.ShapeDtypeStruct((n_pad, 16), jnp.float32),
            jax.ShapeDtypeStruct((n_pad, CP), jnp.float32),
        ],
    )(h_pad, deg16)


def _lsm_body(u_ref, dg_ref, o_ref):
    u = u_ref[...][:, :40]
    deg = dg_ref[...][:, 0:1] + 1.0
    z = u * jnp.sqrt(deg)
    m = jnp.max(z, axis=1, keepdims=True)
    e = jnp.exp(z - m)
    s = jnp.sum(e, axis=1, keepdims=True)
    o_ref[...] = (z - m) - jnp.log(s)


def _log_softmax(uk, deg16, n_pad, blk):
    return pl.pallas_call(
        _lsm_body,
        grid=(n_pad // blk,),
        in_specs=[
            pl.BlockSpec((blk, CP), lambda i: (i, 0)),
            pl.BlockSpec((blk, 16), lambda i: (i, 0)),
        ],
        out_specs=pl.BlockSpec((blk, 40), lambda i: (i, 0)),
        out_shape=jax.ShapeDtypeStruct((n_pad, 40), jnp.float32),
    )(uk, deg16)


def _make_deg(n_pad, e_pad):
    npt = n_pad // NTILES
    ept = e_pad // NTILES
    nch = ept // SUB
    nsc = npt // SUB

    mesh = plsc.VectorSubcoreMesh(
        core_axis_name="c", subcore_axis_name="s", num_cores=1
    )

    @functools.partial(
        pl.kernel,
        mesh=mesh,
        compiler_params=pltpu.CompilerParams(use_tc_tiling_on_sc=False),
        out_type=jax.ShapeDtypeStruct((n_pad, 16), jnp.float32),
        scratch_types=(
            pltpu.VMEM_SHARED((n_pad, 16), jnp.float32),
            pltpu.VMEM((nch, SUB), jnp.int32),
            pltpu.VMEM((SUB, 16), jnp.float32),
            pltpu.VMEM((SUB, 16), jnp.float32),
            pltpu.SemaphoreType.DMA,
            pltpu.SemaphoreType.DMA,
            pltpu.SemaphoreType.DMA,
            pltpu.SemaphoreType.DMA,
        ),
    )
    def deg_kernel(col_hbm, deg16_hbm, deg_s, cibuf, onesb, z16,
                   ssem0, ssem1, ssem2, ssem3):
        ssems = (ssem0, ssem1, ssem2, ssem3)
        sid = lax.axis_index("s")
        nbase = sid * npt

        @pl.loop(0, SUB)
        def _init(n):
            z16[n, pl.ds(0, 16)] = jnp.zeros((LANES,), jnp.float32)
            onesb[n, pl.ds(0, 16)] = jnp.ones((LANES,), jnp.float32)

        pltpu.sync_copy(col_hbm.at[pl.ds(sid * nch, nch)], cibuf)

        @pl.loop(0, nsc)
        def _zero(c2):
            pltpu.sync_copy(z16, deg_s.at[pl.ds(nbase + c2 * SUB, SUB)])

        plsc.subcore_barrier()

        @pl.loop(0, nch // 4)
        def _scat(i4):
            c0 = 4 * i4
            for s in range(4):
                @pl.when(i4 > 0)
                def _(s=s):
                    pltpu.make_async_copy(
                        onesb, deg_s.at[cibuf.at[c0 + s]], ssems[s]).wait()
                pltpu.async_copy(onesb, deg_s.at[cibuf.at[c0 + s]],
                                 ssems[s], add=True)

        for s in range(4):
            pltpu.make_async_copy(onesb, deg_s.at[cibuf.at[s]],
                                  ssems[s]).wait()
        for s in range(nch % 4):
            ct = (nch // 4) * 4 + s
            pltpu.sync_copy(onesb, deg_s.at[cibuf.at[ct]], add=True)

        plsc.subcore_barrier()

        @pl.loop(0, nsc)
        def _out(c2):
            base = nbase + c2 * SUB
            pltpu.sync_copy(deg_s.at[pl.ds(base, SUB)], z16)
            pltpu.sync_copy(z16, deg16_hbm.at[pl.ds(base, SUB)])

    return deg_kernel


def _make_prop(n_pad, e_pad):
    npt = n_pad // NTILES
    ept = e_pad // NTILES
    nch = ept // SUB
    nsc = npt // SUB

    mesh = plsc.VectorSubcoreMesh(
        core_axis_name="c", subcore_axis_name="s", num_cores=1
    )

    @functools.partial(
        pl.kernel,
        mesh=mesh,
        compiler_params=pltpu.CompilerParams(use_tc_tiling_on_sc=False),
        out_type=jax.ShapeDtypeStruct((n_pad, CP), jnp.float32),
        scratch_types=(
            pltpu.VMEM_SHARED((n_pad, CP), jnp.float32),
            pltpu.VMEM((nch, SUB), jnp.int32),
            pltpu.VMEM((nch, SUB), jnp.int32),
            pltpu.VMEM((SUB, CP), jnp.float32),
            pltpu.VMEM((SUB, CP), jnp.float32),
            pltpu.VMEM((SUB, CP), jnp.float32),
            pltpu.VMEM((SUB, CP), jnp.float32),
            pltpu.VMEM((SUB, CP), jnp.float32),
            pltpu.VMEM((SUB, CP), jnp.float32),
            pltpu.VMEM((SUB, CP), jnp.float32),
            pltpu.VMEM((SUB, CP), jnp.float32),
            pltpu.VMEM((npt, 16), jnp.float32),
            pltpu.SemaphoreType.DMA,
            pltpu.SemaphoreType.DMA,
            pltpu.SemaphoreType.DMA,
            pltpu.SemaphoreType.DMA,
            pltpu.SemaphoreType.DMA,
            pltpu.SemaphoreType.DMA,
            pltpu.SemaphoreType.DMA,
            pltpu.SemaphoreType.DMA,
            pltpu.SemaphoreType.DMA,
            pltpu.SemaphoreType.DMA,
            pltpu.SemaphoreType.DMA,
            pltpu.SemaphoreType.DMA,
            pltpu.SemaphoreType.DMA,
            pltpu.SemaphoreType.DMA,
            pltpu.SemaphoreType.DMA,
            pltpu.SemaphoreType.DMA,
        ),
    )
    def prop(u0_hbm, d2_hbm, a0_hbm, row_hbm, col_hbm, uk_hbm,
             acc_s, ribuf, cibuf, gbuf0, gbuf1, gbuf2, gbuf3,
             abuf0, abuf1, ubuf0, ubuf1, d2res,
             gsem0, gsem1, gsem2, gsem3, ssem0, ssem1, ssem2, ssem3,
             lsem0, lsem1, msem0, msem1, wsem0, wsem1, rsem0, rsem1):
        gbufs = (gbuf0, gbuf1, gbuf2, gbuf3)
        gsems = (gsem0, gsem1, gsem2, gsem3)
        ssems = (ssem0, ssem1, ssem2, ssem3)
        abufs = (abuf0, abuf1)
        ubufs = (ubuf0, ubuf1)
        lsems = (lsem0, lsem1)
        msems = (msem0, msem1)
        wsems = (wsem0, wsem1)
        rsems = (rsem0, rsem1)
        sid = lax.axis_index("s")
        nbase = sid * npt

        pltpu.sync_copy(row_hbm.at[pl.ds(sid * nch, nch)], ribuf)
        pltpu.sync_copy(col_hbm.at[pl.ds(sid * nch, nch)], cibuf)
        pltpu.sync_copy(d2_hbm.at[pl.ds(nbase, npt)], d2res)

        @pl.loop(0, nsc)
        def _seed(c2):
            base = nbase + c2 * SUB
            pltpu.sync_copy(a0_hbm.at[pl.ds(base, SUB)],
                            acc_s.at[pl.ds(base, SUB)])
            pltpu.sync_copy(u0_hbm.at[pl.ds(base, SUB)], ubuf0)
            pltpu.sync_copy(ubuf0, uk_hbm.at[pl.ds(base, SUB)])

        plsc.subcore_barrier()

        for _k in range(KHOPS):
            last = _k == KHOPS - 1

            @pl.loop(0, nch // 4)
            def _edge(i4):
                c0 = 4 * i4
                gds = []
                for s in range(4):
                    @pl.when(i4 > 0)
                    def _(s=s):
                        pltpu.make_async_copy(
                            gbufs[s], acc_s.at[cibuf.at[c0 + s]],
                            ssems[s]).wait()
                    gds.append(pltpu.async_copy(
                        uk_hbm.at[ribuf.at[c0 + s]], gbufs[s], gsems[s]))
                for s in range(4):
                    gds[s].wait()
                    pltpu.async_copy(gbufs[s], acc_s.at[cibuf.at[c0 + s]],
                                     ssems[s], add=True)

            for s in range(4):
                pltpu.make_async_copy(gbufs[s], acc_s.at[cibuf.at[s]],
                                      ssems[s]).wait()
            for s in range(nch % 4):
                ct = (nch // 4) * 4 + s
                pltpu.async_copy(uk_hbm.at[ribuf.at[ct]], gbufs[s],
                                 gsems[s]).wait()
                pltpu.sync_copy(gbufs[s], acc_s.at[cibuf.at[ct]], add=True)

            plsc.subcore_barrier()

            def load_chunk(c, s):
                base = nbase + c * SUB
                da = pltpu.async_copy(acc_s.at[pl.ds(base, SUB)],
                                      abufs[s], lsems[s])
                du = pltpu.async_copy(uk_hbm.at[pl.ds(base, SUB)],
                                      ubufs[s], msems[s])
                return da, du

            pend_load = load_chunk(0, 0)
            pend_wb = [None, None]
            for c in range(nsc):
                s = c % 2
                da, du = pend_load
                da.wait()
                du.wait()
                if c + 1 < nsc:
                    s2 = 1 - s
                    if pend_wb[s2] is not None:
                        for d in pend_wb[s2]:
                            d.wait()
                        pend_wb[s2] = None
                    pend_load = load_chunk(c + 1, s2)

                @pl.loop(0, SUB)
                def _comp(n, c=c, s=s):
                    d2v = d2res[c * SUB + n, pl.ds(0, 16)]
                    for j in range(CP // 16):
                        sl = pl.ds(16 * j, 16)
                        ubufs[s][n, sl] = d2v * (abufs[s][n, sl]
                                                 + ubufs[s][n, sl])

                base = nbase + c * SUB
                wb = [pltpu.async_copy(ubufs[s], uk_hbm.at[pl.ds(base, SUB)],
                                       wsems[s])]
                if not last:
                    wb.append(pltpu.async_copy(
                        a0_hbm.at[pl.ds(base, SUB)],
                        acc_s.at[pl.ds(base, SUB)], rsems[s]))
                pend_wb[s] = wb

            for s in range(2):
                if pend_wb[s] is not None:
                    for d in pend_wb[s]:
                        d.wait()

            plsc.subcore_barrier()

    return prop


def kernel(x, edge_index, W1, b1, W2, b2):
    n, _ = x.shape
    e = edge_index.shape[1]
    grain = NTILES * SUB
    n_pad = pl.cdiv(n, grain) * grain
    e_pad = pl.cdiv(e, grain) * grain

    x_pad = jnp.pad(x, ((0, n_pad - n), (0, 0)))
    w1t = W1.T
    w2t = W2.T
    b1r = b1.reshape(1, -1)
    b2r = b2.reshape(1, -1)

    fill = jnp.full((e_pad - e,), n_pad - 1, jnp.int32)
    rowp = jnp.concatenate([edge_index[0].astype(jnp.int32), fill])
    colp = jnp.concatenate([edge_index[1].astype(jnp.int32), fill])
    row2d = rowp.reshape(e_pad // SUB, SUB)
    col2d = colp.reshape(e_pad // SUB, SUB)

    deg16 = _make_deg(n_pad, e_pad)(col2d)
    h_pad = _mlp(x_pad, w1t, b1r, w2t, b2r, n_pad, 256)
    u0, d2, a0 = _prep(h_pad, deg16, n_pad, 256)
    uk = _make_prop(n_pad, e_pad)(u0, d2, a0, row2d, col2d)
    out = _log_softmax(uk, deg16, n_pad, 256)
    return out[:n]


# speedup vs baseline: 1.0207x; 1.0207x over previous
import functools

import jax
import jax.numpy as jnp
from jax import lax
from jax.experimental import pallas as pl
from jax.experimental.pallas import tpu as pltpu
from jax.experimental.pallas import tpu_sc as plsc

ALPHA = 0.1
KHOPS = 10
LANES = 16
NTILES = 16
SUB = 128
CP = 48


def _mlp_body(x_ref, w1_ref, b1_ref, w2_ref, b2_ref, dg_ref,
              u0_ref, d2_ref, a0_ref):
    h = jnp.dot(x_ref[...], w1_ref[...], preferred_element_type=jnp.float32)
    h = jnp.maximum(h + b1_ref[...], 0.0)
    h2 = jnp.dot(h, w2_ref[...], preferred_element_type=jnp.float32)
    h2 = h2 + b2_ref[...]
    pad = jnp.zeros((h2.shape[0], CP - h2.shape[1]), jnp.float32)
    hp = jnp.concatenate([h2, pad], axis=1)
    deg = dg_ref[...][:, 0:1] + 1.0
    dinv = lax.rsqrt(deg)
    u0_ref[...] = dinv * hp
    d2_ref[...] = jnp.broadcast_to((1.0 - ALPHA) * dinv * dinv,
                                   (hp.shape[0], 16))
    a0_ref[...] = (ALPHA / (1.0 - ALPHA)) * hp * jnp.sqrt(deg)


def _mlp_prep(x_pad, w1t, b1r, w2t, b2r, deg16, n_pad, blk):
    f_in = x_pad.shape[1]
    hid = w1t.shape[1]
    c = w2t.shape[1]
    return pl.pallas_call(
        _mlp_body,
        grid=(n_pad // blk,),
        in_specs=[
            pl.BlockSpec((blk, f_in), lambda i: (i, 0)),
            pl.BlockSpec((f_in, hid), lambda i: (0, 0)),
            pl.BlockSpec((1, hid), lambda i: (0, 0)),
            pl.BlockSpec((hid, c), lambda i: (0, 0)),
            pl.BlockSpec((1, c), lambda i: (0, 0)),
            pl.BlockSpec((blk, 16), lambda i: (i, 0)),
        ],
        out_specs=[
            pl.BlockSpec((blk, CP), lambda i: (i, 0)),
            pl.BlockSpec((blk, 16), lambda i: (i, 0)),
            pl.BlockSpec((blk, CP), lambda i: (i, 0)),
        ],
        out_shape=[
            jax.ShapeDtypeStruct((n_pad, CP), jnp.float32),
            jax.ShapeDtypeStruct((n_pad, 16), jnp.float32),
            jax.ShapeDtypeStruct((n_pad, CP), jnp.float32),
        ],
    )(x_pad, w1t, b1r, w2t, b2r, deg16)


def _lsm_body(u_ref, dg_ref, o_ref):
    u = u_ref[...][:, :40]
    deg = dg_ref[...][:, 0:1] + 1.0
    z = u * jnp.sqrt(deg)
    m = jnp.max(z, axis=1, keepdims=True)
    e = jnp.exp(z - m)
    s = jnp.sum(e, axis=1, keepdims=True)
    o_ref[...] = (z - m) - jnp.log(s)


def _log_softmax(uk, deg16, n_pad, blk):
    return pl.pallas_call(
        _lsm_body,
        grid=(n_pad // blk,),
        in_specs=[
            pl.BlockSpec((blk, CP), lambda i: (i, 0)),
            pl.BlockSpec((blk, 16), lambda i: (i, 0)),
        ],
        out_specs=pl.BlockSpec((blk, 40), lambda i: (i, 0)),
        out_shape=jax.ShapeDtypeStruct((n_pad, 40), jnp.float32),
    )(uk, deg16)


def _make_deg(n_pad, e_pad):
    npt = n_pad // NTILES
    ept = e_pad // NTILES
    nch = ept // SUB
    nsc = npt // SUB

    mesh = plsc.VectorSubcoreMesh(
        core_axis_name="c", subcore_axis_name="s", num_cores=1
    )

    @functools.partial(
        pl.kernel,
        mesh=mesh,
        compiler_params=pltpu.CompilerParams(use_tc_tiling_on_sc=False),
        out_type=jax.ShapeDtypeStruct((n_pad, 16), jnp.float32),
        scratch_types=(
            pltpu.VMEM_SHARED((n_pad, 16), jnp.float32),
            pltpu.VMEM((nch, SUB), jnp.int32),
            pltpu.VMEM((SUB, 16), jnp.float32),
            pltpu.VMEM((SUB, 16), jnp.float32),
            pltpu.SemaphoreType.DMA,
            pltpu.SemaphoreType.DMA,
            pltpu.SemaphoreType.DMA,
            pltpu.SemaphoreType.DMA,
        ),
    )
    def deg_kernel(col_hbm, deg16_hbm, deg_s, cibuf, onesb, z16,
                   ssem0, ssem1, ssem2, ssem3):
        ssems = (ssem0, ssem1, ssem2, ssem3)
        sid = lax.axis_index("s")
        nbase = sid * npt

        @pl.loop(0, SUB)
        def _init(n):
            z16[n, pl.ds(0, 16)] = jnp.zeros((LANES,), jnp.float32)
            onesb[n, pl.ds(0, 16)] = jnp.ones((LANES,), jnp.float32)

        pltpu.sync_copy(col_hbm.at[pl.ds(sid * nch, nch)], cibuf)

        @pl.loop(0, nsc)
        def _zero(c2):
            pltpu.sync_copy(z16, deg_s.at[pl.ds(nbase + c2 * SUB, SUB)])

        plsc.subcore_barrier()

        @pl.loop(0, nch // 4)
        def _scat(i4):
            c0 = 4 * i4
            for s in range(4):
                @pl.when(i4 > 0)
                def _(s=s):
                    pltpu.make_async_copy(
                        onesb, deg_s.at[cibuf.at[c0 + s]], ssems[s]).wait()
                pltpu.async_copy(onesb, deg_s.at[cibuf.at[c0 + s]],
                                 ssems[s], add=True)

        for s in range(4):
            pltpu.make_async_copy(onesb, deg_s.at[cibuf.at[s]],
                                  ssems[s]).wait()
        for s in range(nch % 4):
            ct = (nch // 4) * 4 + s
            pltpu.sync_copy(onesb, deg_s.at[cibuf.at[ct]], add=True)

        plsc.subcore_barrier()

        @pl.loop(0, nsc)
        def _out(c2):
            base = nbase + c2 * SUB
            pltpu.sync_copy(deg_s.at[pl.ds(base, SUB)], z16)
            pltpu.sync_copy(z16, deg16_hbm.at[pl.ds(base, SUB)])

    return deg_kernel


def _make_prop(n_pad, e_pad):
    npt = n_pad // NTILES
    ept = e_pad // NTILES
    nch = ept // SUB
    nsc = npt // SUB

    mesh = plsc.VectorSubcoreMesh(
        core_axis_name="c", subcore_axis_name="s", num_cores=1
    )

    @functools.partial(
        pl.kernel,
        mesh=mesh,
        compiler_params=pltpu.CompilerParams(use_tc_tiling_on_sc=False),
        out_type=jax.ShapeDtypeStruct((n_pad, CP), jnp.float32),
        scratch_types=(
            pltpu.VMEM_SHARED((n_pad, CP), jnp.float32),
            pltpu.VMEM((nch, SUB), jnp.int32),
            pltpu.VMEM((nch, SUB), jnp.int32),
            pltpu.VMEM((SUB, CP), jnp.float32),
            pltpu.VMEM((SUB, CP), jnp.float32),
            pltpu.VMEM((SUB, CP), jnp.float32),
            pltpu.VMEM((SUB, CP), jnp.float32),
            pltpu.VMEM((SUB, CP), jnp.float32),
            pltpu.VMEM((SUB, CP), jnp.float32),
            pltpu.VMEM((SUB, CP), jnp.float32),
            pltpu.VMEM((SUB, CP), jnp.float32),
            pltpu.VMEM((npt, 16), jnp.float32),
            pltpu.SemaphoreType.DMA,
            pltpu.SemaphoreType.DMA,
            pltpu.SemaphoreType.DMA,
            pltpu.SemaphoreType.DMA,
            pltpu.SemaphoreType.DMA,
            pltpu.SemaphoreType.DMA,
            pltpu.SemaphoreType.DMA,
            pltpu.SemaphoreType.DMA,
            pltpu.SemaphoreType.DMA,
            pltpu.SemaphoreType.DMA,
            pltpu.SemaphoreType.DMA,
            pltpu.SemaphoreType.DMA,
            pltpu.SemaphoreType.DMA,
            pltpu.SemaphoreType.DMA,
            pltpu.SemaphoreType.DMA,
            pltpu.SemaphoreType.DMA,
        ),
    )
    def prop(u0_hbm, d2_hbm, a0_hbm, row_hbm, col_hbm, uk_hbm,
             acc_s, ribuf, cibuf, gbuf0, gbuf1, gbuf2, gbuf3,
             abuf0, abuf1, ubuf0, ubuf1, d2res,
             gsem0, gsem1, gsem2, gsem3, ssem0, ssem1, ssem2, ssem3,
             lsem0, lsem1, msem0, msem1, wsem0, wsem1, rsem0, rsem1):
        gbufs = (gbuf0, gbuf1, gbuf2, gbuf3)
        gsems = (gsem0, gsem1, gsem2, gsem3)
        ssems = (ssem0, ssem1, ssem2, ssem3)
        abufs = (abuf0, abuf1)
        ubufs = (ubuf0, ubuf1)
        lsems = (lsem0, lsem1)
        msems = (msem0, msem1)
        wsems = (wsem0, wsem1)
        rsems = (rsem0, rsem1)
        sid = lax.axis_index("s")
        nbase = sid * npt

        pltpu.sync_copy(row_hbm.at[pl.ds(sid * nch, nch)], ribuf)
        pltpu.sync_copy(col_hbm.at[pl.ds(sid * nch, nch)], cibuf)
        pltpu.sync_copy(d2_hbm.at[pl.ds(nbase, npt)], d2res)

        @pl.loop(0, nsc)
        def _seed(c2):
            base = nbase + c2 * SUB
            pltpu.sync_copy(a0_hbm.at[pl.ds(base, SUB)],
                            acc_s.at[pl.ds(base, SUB)])
            pltpu.sync_copy(u0_hbm.at[pl.ds(base, SUB)], ubuf0)
            pltpu.sync_copy(ubuf0, uk_hbm.at[pl.ds(base, SUB)])

        plsc.subcore_barrier()

        for _k in range(KHOPS):
            last = _k == KHOPS - 1

            @pl.loop(0, nch // 4)
            def _edge(i4):
                c0 = 4 * i4
                gds = []
                for s in range(4):
                    @pl.when(i4 > 0)
                    def _(s=s):
                        pltpu.make_async_copy(
                            gbufs[s], acc_s.at[cibuf.at[c0 + s]],
                            ssems[s]).wait()
                    gds.append(pltpu.async_copy(
                        uk_hbm.at[ribuf.at[c0 + s]], gbufs[s], gsems[s]))
                for s in range(4):
                    gds[s].wait()
                    pltpu.async_copy(gbufs[s], acc_s.at[cibuf.at[c0 + s]],
                                     ssems[s], add=True)

            for s in range(4):
                pltpu.make_async_copy(gbufs[s], acc_s.at[cibuf.at[s]],
                                      ssems[s]).wait()
            for s in range(nch % 4):
                ct = (nch // 4) * 4 + s
                pltpu.async_copy(uk_hbm.at[ribuf.at[ct]], gbufs[s],
                                 gsems[s]).wait()
                pltpu.sync_copy(gbufs[s], acc_s.at[cibuf.at[ct]], add=True)

            plsc.subcore_barrier()

            def load_chunk(c, s):
                base = nbase + c * SUB
                da = pltpu.async_copy(acc_s.at[pl.ds(base, SUB)],
                                      abufs[s], lsems[s])
                du = pltpu.async_copy(uk_hbm.at[pl.ds(base, SUB)],
                                      ubufs[s], msems[s])
                return da, du

            pend_load = load_chunk(0, 0)
            pend_wb = [None, None]
            for c in range(nsc):
                s = c % 2
                da, du = pend_load
                da.wait()
                du.wait()
                if c + 1 < nsc:
                    s2 = 1 - s
                    if pend_wb[s2] is not None:
                        for d in pend_wb[s2]:
                            d.wait()
                        pend_wb[s2] = None
                    pend_load = load_chunk(c + 1, s2)

                @plsc.parallel_loop(0, SUB, unroll=4)
                def _comp(n, c=c, s=s):
                    d2v = d2res[c * SUB + n, pl.ds(0, 16)]
                    for j in range(CP // 16):
                        sl = pl.ds(16 * j, 16)
                        ubufs[s][n, sl] = d2v * (abufs[s][n, sl]
                                                 + ubufs[s][n, sl])

                base = nbase + c * SUB
                wb = [pltpu.async_copy(ubufs[s], uk_hbm.at[pl.ds(base, SUB)],
                                       wsems[s])]
                if not last:
                    wb.append(pltpu.async_copy(
                        a0_hbm.at[pl.ds(base, SUB)],
                        acc_s.at[pl.ds(base, SUB)], rsems[s]))
                pend_wb[s] = wb

            for s in range(2):
                if pend_wb[s] is not None:
                    for d in pend_wb[s]:
                        d.wait()

            plsc.subcore_barrier()

    return prop


def kernel(x, edge_index, W1, b1, W2, b2):
    n, _ = x.shape
    e = edge_index.shape[1]
    grain = NTILES * SUB
    n_pad = pl.cdiv(n, grain) * grain
    e_pad = pl.cdiv(e, grain) * grain

    x_pad = jnp.pad(x, ((0, n_pad - n), (0, 0)))
    w1t = W1.T
    w2t = W2.T
    b1r = b1.reshape(1, -1)
    b2r = b2.reshape(1, -1)

    fill = jnp.full((e_pad - e,), n_pad - 1, jnp.int32)
    rowp = jnp.concatenate([edge_index[0].astype(jnp.int32), fill])
    colp = jnp.concatenate([edge_index[1].astype(jnp.int32), fill])
    row2d = rowp.reshape(e_pad // SUB, SUB)
    col2d = colp.reshape(e_pad // SUB, SUB)

    deg16 = _make_deg(n_pad, e_pad)(col2d)
    u0, d2, a0 = _mlp_prep(x_pad, w1t, b1r, w2t, b2r, deg16, n_pad, 256)
    uk = _make_prop(n_pad, e_pad)(u0, d2, a0, row2d, col2d)
    out = _log_softmax(uk, deg16, n_pad, 256)
    return out[:n]
